# baseline (speedup 1.0000x reference)
import functools

import jax
import jax.numpy as jnp
from jax import lax
from jax.experimental import pallas as pl
from jax.experimental.pallas import tpu as pltpu
from jax.experimental.pallas import tpu_sc as plsc

EMBED_DIM = 4096
BATCH = 1024

NC = 2
NS = 16
NW = NC * NS
B_PER_W = BATCH // NW
CHUNK = 8
NCHUNK = B_PER_W // CHUNK

_mesh = plsc.VectorSubcoreMesh(core_axis_name="c", subcore_axis_name="s")

_out_struct = jax.ShapeDtypeStruct((BATCH, EMBED_DIM), jnp.float32)


@functools.partial(
    pl.kernel,
    mesh=_mesh,
    out_type=(_out_struct, _out_struct),
    scratch_types=[
        pltpu.VMEM((B_PER_W, 20), jnp.int32),
        pltpu.VMEM((B_PER_W,), jnp.int32),
        pltpu.VMEM((CHUNK, EMBED_DIM), jnp.float32),
        pltpu.VMEM((CHUNK, EMBED_DIM), jnp.float32),
        pltpu.VMEM((CHUNK, EMBED_DIM), jnp.float32),
        pltpu.SemaphoreType.DMA,
        pltpu.SemaphoreType.DMA,
        pltpu.SemaphoreType.DMA,
        pltpu.SemaphoreType.DMA,
        pltpu.SemaphoreType.DMA,
        pltpu.SemaphoreType.DMA,
        pltpu.SemaphoreType.DMA,
        pltpu.SemaphoreType.DMA,
    ],
)
def _cls_gather(x_hbm, table_hbm, out1_hbm, out2_hbm, x_v, idx_v,
                buf0, buf1, buf2, sg0, sg1, sg2, sw0, sw1, sw2, sc0, sc1):
    wid = lax.axis_index("s") * NC + lax.axis_index("c")
    base = wid * B_PER_W
    xc0 = pltpu.async_copy(x_hbm.at[pl.ds(base, 16)], x_v.at[pl.ds(0, 16)], sc0)
    xc1 = pltpu.async_copy(x_hbm.at[pl.ds(base + 16, 16)],
                           x_v.at[pl.ds(16, 16)], sc1)
    lanes = lax.iota(jnp.int32, 16)

    def _extract_half(half):
        acc = jnp.zeros((16,), jnp.int32)
        for i in range(16):
            row = x_v[half * 16 + i, pl.ds(0, 16)]
            acc = jnp.where(lanes == i, row[0], acc)
        idx_v[pl.ds(half * 16, 16)] = acc

    xc0.wait()
    _extract_half(0)
    g0 = pltpu.async_copy(table_hbm.at[idx_v.at[pl.ds(0, CHUNK)]], buf0, sg0)
    g1 = pltpu.async_copy(table_hbm.at[idx_v.at[pl.ds(CHUNK, CHUNK)]], buf1, sg1)
    xc1.wait()
    _extract_half(1)
    g2 = pltpu.async_copy(table_hbm.at[idx_v.at[pl.ds(2 * CHUNK, CHUNK)]], buf2, sg2)
    g0.wait()
    wa0 = pltpu.async_copy(buf0, out1_hbm.at[pl.ds(base, CHUNK)], sw0)
    wb0 = pltpu.async_copy(buf0, out2_hbm.at[pl.ds(base, CHUNK)], sw0)
    g1.wait()
    wa1 = pltpu.async_copy(buf1, out1_hbm.at[pl.ds(base + CHUNK, CHUNK)], sw1)
    wb1 = pltpu.async_copy(buf1, out2_hbm.at[pl.ds(base + CHUNK, CHUNK)], sw1)
    g2.wait()
    wa2 = pltpu.async_copy(buf2, out1_hbm.at[pl.ds(base + 2 * CHUNK, CHUNK)], sw2)
    wb2 = pltpu.async_copy(buf2, out2_hbm.at[pl.ds(base + 2 * CHUNK, CHUNK)], sw2)
    wa0.wait()
    wb0.wait()
    g3 = pltpu.async_copy(table_hbm.at[idx_v.at[pl.ds(3 * CHUNK, CHUNK)]], buf0, sg0)
    g3.wait()
    wa3 = pltpu.async_copy(buf0, out1_hbm.at[pl.ds(base + 3 * CHUNK, CHUNK)], sw0)
    wb3 = pltpu.async_copy(buf0, out2_hbm.at[pl.ds(base + 3 * CHUNK, CHUNK)], sw0)
    wa1.wait()
    wb1.wait()
    wa2.wait()
    wb2.wait()
    wa3.wait()
    wb3.wait()


def kernel(x, table):
    out1, out2 = _cls_gather(x, table)
    return (out1, out2)


# speedup vs baseline: 1.0114x; 1.0114x over previous
import functools

import jax
import jax.numpy as jnp
from jax import lax
from jax.experimental import pallas as pl
from jax.experimental.pallas import tpu as pltpu
from jax.experimental.pallas import tpu_sc as plsc

EMBED_DIM = 4096
BATCH = 1024
SEQLEN = 20

NC = 2
NS = 16
NW = NC * NS
B_PER_W = BATCH // NW
CHUNK = 8

_mesh = plsc.VectorSubcoreMesh(core_axis_name="c", subcore_axis_name="s")

_out_struct = jax.ShapeDtypeStruct((BATCH, EMBED_DIM), jnp.float32)


@functools.partial(
    pl.kernel,
    mesh=_mesh,
    out_type=(_out_struct, _out_struct),
    scratch_types=[
        pltpu.VMEM((B_PER_W,), jnp.int32),
        pltpu.VMEM((B_PER_W,), jnp.int32),
        pltpu.VMEM((CHUNK, EMBED_DIM), jnp.float32),
        pltpu.VMEM((CHUNK, EMBED_DIM), jnp.float32),
        pltpu.VMEM((CHUNK, EMBED_DIM), jnp.float32),
        pltpu.SemaphoreType.DMA,
        pltpu.SemaphoreType.DMA,
        pltpu.SemaphoreType.DMA,
        pltpu.SemaphoreType.DMA,
        pltpu.SemaphoreType.DMA,
        pltpu.SemaphoreType.DMA,
    ],
)
def _cls_gather(x_hbm, table_hbm, out1_hbm, out2_hbm, off_v, idx_v,
                buf0, buf1, buf2, sg0, sg1, sg2, sw0, sw1, sw2):
    wid = lax.axis_index("s") * NC + lax.axis_index("c")
    base = wid * B_PER_W
    strided = lax.iota(jnp.int32, 16) * SEQLEN + base * SEQLEN
    off_v[pl.ds(0, 16)] = strided
    off_v[pl.ds(16, 16)] = strided + 16 * SEQLEN
    pltpu.async_copy(x_hbm.at[off_v], idx_v, sg0).wait()
    g0 = pltpu.async_copy(table_hbm.at[idx_v.at[pl.ds(0, CHUNK)]], buf0, sg0)
    g1 = pltpu.async_copy(table_hbm.at[idx_v.at[pl.ds(CHUNK, CHUNK)]], buf1, sg1)
    g2 = pltpu.async_copy(table_hbm.at[idx_v.at[pl.ds(2 * CHUNK, CHUNK)]], buf2, sg2)
    g0.wait()
    wa0 = pltpu.async_copy(buf0, out1_hbm.at[pl.ds(base, CHUNK)], sw0)
    wb0 = pltpu.async_copy(buf0, out2_hbm.at[pl.ds(base, CHUNK)], sw0)
    g1.wait()
    wa1 = pltpu.async_copy(buf1, out1_hbm.at[pl.ds(base + CHUNK, CHUNK)], sw1)
    wb1 = pltpu.async_copy(buf1, out2_hbm.at[pl.ds(base + CHUNK, CHUNK)], sw1)
    g2.wait()
    wa2 = pltpu.async_copy(buf2, out1_hbm.at[pl.ds(base + 2 * CHUNK, CHUNK)], sw2)
    wb2 = pltpu.async_copy(buf2, out2_hbm.at[pl.ds(base + 2 * CHUNK, CHUNK)], sw2)
    wa0.wait()
    wb0.wait()
    g3 = pltpu.async_copy(table_hbm.at[idx_v.at[pl.ds(3 * CHUNK, CHUNK)]], buf0, sg0)
    g3.wait()
    wa3 = pltpu.async_copy(buf0, out1_hbm.at[pl.ds(base + 3 * CHUNK, CHUNK)], sw0)
    wb3 = pltpu.async_copy(buf0, out2_hbm.at[pl.ds(base + 3 * CHUNK, CHUNK)], sw0)
    wa1.wait()
    wb1.wait()
    wa2.wait()
    wb2.wait()
    wa3.wait()
    wb3.wait()


def kernel(x, table):
    out1, out2 = _cls_gather(x.reshape(-1), table)
    return (out1, out2)
